# baseline (speedup 1.0000x reference)
import functools

import jax
import jax.numpy as jnp
import numpy as np
from jax import lax
from jax.experimental import pallas as pl
from jax.experimental.pallas import tpu as pltpu
from jax.experimental.pallas import tpu_sc as plsc

B, S, D = 4, 4096, 2048
MASK_PCT = 0.15
S_BLK = 512
T = S // S_BLK


def _rotl32(x, r):
    r = np.uint32(r)
    return (x << r) | (x >> np.uint32(32 - r))


def _threefry2x32(ks0, ks1, x0, x1):
    ks2 = ks0 ^ ks1 ^ np.uint32(0x1BD11BDA)
    ks = [ks0, ks1, ks2]
    x0 = (x0 + ks0).astype(np.uint32)
    x1 = (x1 + ks1).astype(np.uint32)
    rot = [[13, 15, 26, 6], [17, 29, 16, 24]]
    for i in range(5):
        for r in rot[i % 2]:
            x0 = (x0 + x1).astype(np.uint32)
            x1 = _rotl32(x1, r)
            x1 = x0 ^ x1
        x0 = (x0 + ks[(i + 1) % 3]).astype(np.uint32)
        x1 = (x1 + ks[(i + 2) % 3] + np.uint32(i + 1)).astype(np.uint32)
    return x0, x1


def _tf_counts(k0, k1, n):
    c = np.arange(n, dtype=np.uint64)
    return _threefry2x32(
        k0, k1, (c >> np.uint64(32)).astype(np.uint32), c.astype(np.uint32)
    )


def _np_uniform(k0, k1, shape):
    o0, o1 = _tf_counts(k0, k1, int(np.prod(shape)))
    bits = o0 ^ o1
    u = ((bits >> np.uint32(9)) | np.uint32(0x3F800000)).view(np.float32)
    return (u - np.float32(1.0)).reshape(shape)


_c0, _c1 = _tf_counts(np.uint32(0), np.uint32(1), 2)
_scores_ne = _np_uniform(_c0[0], _c1[0], (B, S))
_scores_all = _np_uniform(_c0[1], _c1[1], (B, S))
_perm = np.argsort(_scores_ne, axis=1, kind="stable").astype(np.int32)
_ranks_all = np.argsort(
    np.argsort(_scores_all, axis=1, kind="stable"), axis=1, kind="stable"
).astype(np.int32)
PERM_GLOB = (_perm + (np.arange(B, dtype=np.int32) * S)[:, None]).reshape(-1)
RA_PERM = np.take_along_axis(_ranks_all, _perm, axis=1).reshape(-1)
del _c0, _c1, _scores_ne, _scores_all

NT = 16
RPC = B // 2
CHUNK = S // NT
NV = CHUNK // 16
GUARD = 8


def _sc_select(ne_hbm, perm_hbm, rap_hbm, mask_hbm, ne_v, idx_v, rap_v,
               vals_v, acc_v, part_v, shared, sem):
    core = lax.axis_index("c")
    tid = lax.axis_index("s")
    base = tid * CHUNK
    for r in range(RPC):
        off = (core + 2 * r) * jnp.int32(S)
        pltpu.sync_copy(ne_hbm.at[pl.ds(off, S)], ne_v)
        pltpu.sync_copy(perm_hbm.at[pl.ds(off + base, CHUNK)], idx_v)
        pltpu.sync_copy(rap_hbm.at[pl.ds(off + base, CHUNK)], rap_v)
        acc = jnp.zeros((16,), jnp.float32)
        for j in range(NV):
            gi = idx_v[pl.ds(j * 16, 16)] - off
            g = plsc.load_gather(ne_v, [gi])
            acc = acc + g
        acc_v[...] = acc
        pltpu.sync_copy(acc_v, shared.at[GUARD + r * NT + tid])
        plsc.subcore_barrier()
        pltpu.sync_copy(shared.at[pl.ds(GUARD + r * NT, NT)], part_v)
        tot = jnp.zeros((16,), jnp.float32)
        pre = jnp.zeros((16,), jnp.float32)
        zero = jnp.zeros((16,), jnp.float32)
        for j in range(NT):
            row = part_v[j]
            tot = tot + row
            pre = pre + jnp.where(jnp.int32(j) < tid, row, zero)
        count = jnp.broadcast_to(jnp.sum(tot), (16,))
        carry = jnp.broadcast_to(jnp.sum(pre), (16,))
        k_ne = (count * jnp.float32(MASK_PCT)).astype(jnp.int32).astype(
            jnp.float32)
        k_e = ((jnp.float32(S) - count) * jnp.float32(0.1)).astype(jnp.int32)
        for j in range(NV):
            gi = idx_v[pl.ds(j * 16, 16)] - off
            g = plsc.load_gather(ne_v, [gi])
            inc = plsc.cumsum(g)
            excl = carry + inc - g
            sel = jnp.where((g > 0.0) & (excl < k_ne), jnp.float32(1.0),
                            jnp.float32(0.0))
            carry = carry + jnp.broadcast_to(jnp.sum(g), (16,))
            rap = rap_v[pl.ds(j * 16, 16)]
            me = jnp.where(rap < k_e, jnp.float32(1.0), jnp.float32(0.0))
            vals_v[pl.ds(j * 16, 16)] = jnp.maximum(sel, me)
        pltpu.async_copy(vals_v, mask_hbm.at[idx_v], sem).wait()


def _mask2d_sc(ne_flat):
    mesh = plsc.VectorSubcoreMesh(core_axis_name="c", subcore_axis_name="s")
    kfn = pl.kernel(
        _sc_select,
        out_type=jax.ShapeDtypeStruct((B * S,), jnp.float32),
        mesh=mesh,
        scratch_types=[
            pltpu.VMEM((S,), jnp.float32),
            pltpu.VMEM((CHUNK,), jnp.int32),
            pltpu.VMEM((CHUNK,), jnp.int32),
            pltpu.VMEM((CHUNK,), jnp.float32),
            pltpu.VMEM((16,), jnp.float32),
            pltpu.VMEM((NT, 16), jnp.float32),
            pltpu.VMEM_SHARED((GUARD + RPC * NT, 16), jnp.float32),
            pltpu.SemaphoreType.DMA,
        ],
        compiler_params=pltpu.CompilerParams(needs_layout_passes=False),
    )
    return kfn(ne_flat, jnp.asarray(PERM_GLOB), jnp.asarray(RA_PERM))


def _reduce_body(x_ref, ne_ref):
    x = x_ref[...]
    ne = jnp.any(x != 0.0, axis=-1)
    ne_ref[...] = ne[:, None, :].astype(jnp.float32)


def _nonempty_flags(data):
    return pl.pallas_call(
        _reduce_body,
        grid=(B, T),
        in_specs=[pl.BlockSpec((1, S_BLK, D), lambda b, t: (b, t, 0))],
        out_specs=pl.BlockSpec((1, 1, S_BLK), lambda b, t: (b, 0, t)),
        out_shape=jax.ShapeDtypeStruct((B, 1, S), jnp.float32),
    )(data).reshape(B * S)


def _apply_body(x_ref, m_ref, out_ref, mask_ref):
    x = x_ref[...]
    m = m_ref[...]
    mb = jnp.broadcast_to(m, x.shape)
    out_ref[...] = (1.0 - mb) * x
    mask_ref[...] = mb


def _apply_mask(data, mask2d):
    m3 = mask2d.reshape(B, S, 1)
    return pl.pallas_call(
        _apply_body,
        grid=(B, T),
        in_specs=[
            pl.BlockSpec((1, S_BLK, D), lambda b, t: (b, t, 0)),
            pl.BlockSpec((1, S_BLK, 1), lambda b, t: (b, t, 0)),
        ],
        out_specs=[
            pl.BlockSpec((1, S_BLK, D), lambda b, t: (b, t, 0)),
            pl.BlockSpec((1, S_BLK, D), lambda b, t: (b, t, 0)),
        ],
        out_shape=[
            jax.ShapeDtypeStruct((B, S, D), jnp.float32),
            jax.ShapeDtypeStruct((B, S, D), jnp.float32),
        ],
    )(data, m3)


def kernel(data):
    ne = _nonempty_flags(data)
    mask2d = _mask2d_sc(ne)
    return _apply_mask(data, mask2d)


# speedup vs baseline: 1.9720x; 1.9720x over previous
import jax
import jax.numpy as jnp
import numpy as np
from jax.experimental import pallas as pl
from jax.experimental.pallas import tpu as pltpu

B, S, D = 4, 4096, 2048
MASK_PCT = 0.15
S_BLK = 512
T = S // S_BLK


def _rotl32(x, r):
    r = np.uint32(r)
    return (x << r) | (x >> np.uint32(32 - r))


def _threefry2x32(ks0, ks1, x0, x1):
    ks2 = ks0 ^ ks1 ^ np.uint32(0x1BD11BDA)
    ks = [ks0, ks1, ks2]
    x0 = (x0 + ks0).astype(np.uint32)
    x1 = (x1 + ks1).astype(np.uint32)
    rot = [[13, 15, 26, 6], [17, 29, 16, 24]]
    for i in range(5):
        for r in rot[i % 2]:
            x0 = (x0 + x1).astype(np.uint32)
            x1 = _rotl32(x1, r)
            x1 = x0 ^ x1
        x0 = (x0 + ks[(i + 1) % 3]).astype(np.uint32)
        x1 = (x1 + ks[(i + 2) % 3] + np.uint32(i + 1)).astype(np.uint32)
    return x0, x1


def _tf_counts(k0, k1, n):
    c = np.arange(n, dtype=np.uint64)
    return _threefry2x32(
        k0, k1, (c >> np.uint64(32)).astype(np.uint32), c.astype(np.uint32)
    )


def _np_uniform(k0, k1, shape):
    o0, o1 = _tf_counts(k0, k1, int(np.prod(shape)))
    bits = o0 ^ o1
    u = ((bits >> np.uint32(9)) | np.uint32(0x3F800000)).view(np.float32)
    return (u - np.float32(1.0)).reshape(shape)


def _ranks_t(scores):
    r = np.argsort(
        np.argsort(scores, axis=1, kind="stable"), axis=1, kind="stable"
    ).astype(np.int32)
    return np.ascontiguousarray(r.reshape(B, T, S_BLK).transpose(0, 2, 1))


_c0, _c1 = _tf_counts(np.uint32(0), np.uint32(1), 2)
RANK_BASE_T = _ranks_t(_np_uniform(_c0[0], _c1[0], (B, S)))
RANKS_ALL_T = _ranks_t(_np_uniform(_c0[1], _c1[1], (B, S)))
del _c0, _c1


def _body(x_ref, rb_ref, ra_ref, out_ref, mask_ref, data_scr, ne_scr, m_scr):
    i = pl.program_id(0)
    t = pl.program_id(1)
    par = jax.lax.rem(i, 2)
    q = jax.lax.rem(i + 1, 2)
    lane2 = jax.lax.broadcasted_iota(jnp.int32, (S_BLK, 2 * T), 1)

    @pl.when((i > 0) & (t == 0))
    def _select():
        colq = (lane2 // T == q).astype(jnp.float32)
        ne = ne_scr[...] * colq
        rank_base = jnp.concatenate([rb_ref[0]] * 2, axis=1)
        ranks_all = jnp.concatenate([ra_ref[0]] * 2, axis=1)
        count = jnp.sum(ne)
        k_ne = (count * MASK_PCT).astype(jnp.int32)
        k_e = ((S - count) * 0.1).astype(jnp.int32)

        k_ne_f = k_ne.astype(jnp.float32)

        def bis(_, lh):
            lo, hi = lh
            mid = (lo + hi) // 2
            n = jnp.sum(ne * (rank_base < mid).astype(jnp.float32))
            pred = n >= k_ne_f
            return (jnp.where(pred, lo, mid + 1), jnp.where(pred, mid, hi))

        lo, _ = jax.lax.fori_loop(
            0, 13, bis, (jnp.int32(0), jnp.int32(S)), unroll=True
        )
        m_scr[...] = jnp.maximum(
            ne * (rank_base < lo).astype(jnp.float32),
            (ranks_all < k_e).astype(jnp.float32) * colq,
        )

    @pl.when(i > 0)
    def _apply():
        xm = data_scr[pl.ds(t * S_BLK, S_BLK), :]
        m = jnp.sum(m_scr[...] * (lane2 == q * T + t), axis=1, keepdims=True)
        mb = jnp.broadcast_to(m, (S_BLK, D))
        out_ref[...] = ((1.0 - mb) * xm)[None]
        mask_ref[...] = mb[None]

    @pl.when(i < B)
    def _load():
        x = x_ref[0]
        data_scr[pl.ds(t * S_BLK, S_BLK), :] = x
        ne = jnp.any(x != 0.0, axis=-1).astype(jnp.float32)
        ne_scr[...] = jnp.where(lane2 == par * T + t, ne[:, None], ne_scr[...])


def kernel(data):
    sel = lambda c, a, b: jax.lax.select(c, jnp.int32(a), jnp.int32(b))
    x_map = lambda i, t: (jnp.minimum(i, B - 1), sel(i < B, t, T - 1), 0)
    c_map = lambda i, t: (jnp.maximum(i - 1, 0), 0, 0)
    out_map = lambda i, t: (jnp.maximum(i - 1, 0), sel(i > 0, t, 0), 0)
    return pl.pallas_call(
        _body,
        grid=(B + 1, T),
        in_specs=[
            pl.BlockSpec((1, S_BLK, D), x_map),
            pl.BlockSpec((1, S_BLK, T), c_map),
            pl.BlockSpec((1, S_BLK, T), c_map),
        ],
        out_specs=[
            pl.BlockSpec((1, S_BLK, D), out_map),
            pl.BlockSpec((1, S_BLK, D), out_map),
        ],
        out_shape=[
            jax.ShapeDtypeStruct((B, S, D), jnp.float32),
            jax.ShapeDtypeStruct((B, S, D), jnp.float32),
        ],
        scratch_shapes=[
            pltpu.VMEM((S, D), jnp.float32),
            pltpu.VMEM((S_BLK, 2 * T), jnp.float32),
            pltpu.VMEM((S_BLK, 2 * T), jnp.float32),
        ],
    )(data, jnp.asarray(RANK_BASE_T), jnp.asarray(RANKS_ALL_T))
